# baseline (speedup 1.0000x reference)
import functools
import math

import jax
import jax.numpy as jnp
from jax import lax
from jax.experimental import pallas as pl
from jax.experimental.pallas import tpu as pltpu
from jax.experimental.pallas import tpu_sc as plsc

_EMBED_DIM = 1024
_PADDING_IDX = 1
_FRONT_PAD = 8 - (_PADDING_IDX + 1)


def _build_table(num_embeddings: int, embed_dim: int, padding_idx: int):
    half = embed_dim // 2
    scale = math.log(10000.0) / (half - 1)
    inv = jnp.exp(jnp.arange(half, dtype=jnp.float32) * -scale)
    pos = jnp.arange(num_embeddings, dtype=jnp.float32)
    ang = pos[:, None] * inv[None, :]
    emb = jnp.concatenate([jnp.sin(ang), jnp.cos(ang)], axis=1)
    emb = emb.at[padding_idx, :].set(0.0)
    return emb


def kernel(x):
    bsz, seq_len = x.shape
    n_rows = bsz * seq_len
    table = _build_table(_PADDING_IDX + 1 + seq_len, _EMBED_DIM, _PADDING_IDX)
    table = jnp.concatenate(
        [jnp.zeros((_FRONT_PAD, _EMBED_DIM), jnp.float32), table], axis=0
    )
    xf = x.reshape(n_rows)

    info = plsc.get_sparse_core_info()
    nc, ns, lanes = info.num_cores, info.num_subcores, info.num_lanes
    nw = nc * ns
    t_per_w = seq_len // nw
    chunk = 32
    n_chunks = t_per_w // chunk
    ring = 3

    mesh = plsc.VectorSubcoreMesh(core_axis_name="c", subcore_axis_name="s")

    @functools.partial(
        pl.kernel,
        mesh=mesh,
        out_type=jax.ShapeDtypeStruct((n_rows, _EMBED_DIM), jnp.float32),
        scratch_types=[
            pltpu.VMEM((bsz, t_per_w), jnp.int32),
            pltpu.VMEM((chunk,), jnp.int32),
            pltpu.VMEM((ring, chunk, _EMBED_DIM), jnp.float32),
            pltpu.SemaphoreType.DMA,
            pltpu.SemaphoreType.DMA,
        ],
    )
    def sc_kernel(table_hbm, x_hbm, out_hbm, xv, idxv, fbuf, sem, fsem):
        wid = lax.axis_index("s") * nc + lax.axis_index("c")
        t0w = wid * t_per_w
        perms = [lax.iota(jnp.int32, lanes) ^ s for s in (1, 2, 4, 8)]

        for b in range(bsz):
            pltpu.sync_copy(x_hbm.at[pl.ds(b * seq_len + t0w, t_per_w)], xv.at[b])

        def drain_one(dst_slice):
            pltpu.make_async_copy(
                table_hbm.at[pl.ds(_PADDING_IDX + 1 + _FRONT_PAD, chunk)],
                dst_slice,
                sem,
            ).wait()

        k = 0
        dst_ring = []
        for ci in range(n_chunks):
            tb = ci * chunk
            for b in range(bsz):
                dst = out_hbm.at[pl.ds(b * seq_len + t0w + tb, chunk)]
                if k >= ring:
                    drain_one(dst_ring[k - ring])
                dst_ring.append(dst)

                acc = None
                for i in range(chunk // lanes):
                    toks = xv[b, pl.ds(tb + i * lanes, lanes)]
                    m = jnp.where(toks == _PADDING_IDX, 1, 0)
                    acc = m if acc is None else acc | m
                for perm in perms:
                    acc = jnp.maximum(
                        acc, acc.at[perm].get(mode="promise_in_bounds")
                    )
                has_pad = acc[0] > 0

                def common(dst=dst, tb=tb):
                    src = table_hbm.at[
                        pl.ds(t0w + tb + _PADDING_IDX + 1 + _FRONT_PAD, chunk)
                    ]
                    pltpu.make_async_copy(src, dst, sem).start()

                def fallback(dst=dst, tb=tb, b=b, slot=k % ring):
                    for i in range(chunk // lanes):
                        toks = xv[b, pl.ds(tb + i * lanes, lanes)]
                        seq_pos = lax.iota(jnp.int32, lanes) + (
                            t0w + tb + i * lanes + _PADDING_IDX + 1 + _FRONT_PAD
                        )
                        idxv[pl.ds(i * lanes, lanes)] = jnp.where(
                            toks != _PADDING_IDX, seq_pos, _PADDING_IDX + _FRONT_PAD
                        )
                    pltpu.async_copy(table_hbm.at[idxv], fbuf.at[slot], fsem).wait()
                    pltpu.make_async_copy(fbuf.at[slot], dst, sem).start()

                lax.cond(has_pad, fallback, common)
                k += 1

        for j in range(min(ring, k)):
            drain_one(dst_ring[k - min(ring, k) + j])

    out = sc_kernel(table, xf)
    return out.reshape(bsz, seq_len, _EMBED_DIM)


# speedup vs baseline: 12.9914x; 12.9914x over previous
import functools
import math

import jax
import jax.numpy as jnp
from jax import lax
from jax.experimental import pallas as pl
from jax.experimental.pallas import tpu as pltpu
from jax.experimental.pallas import tpu_sc as plsc

_EMBED_DIM = 1024
_PADDING_IDX = 1
_FRONT_PAD = 8 - (_PADDING_IDX + 1)
_ZERO_ROW = _PADDING_IDX + _FRONT_PAD


def _build_table(num_embeddings: int, embed_dim: int, padding_idx: int):
    half = embed_dim // 2
    scale = math.log(10000.0) / (half - 1)
    inv = jnp.exp(jnp.arange(half, dtype=jnp.float32) * -scale)
    pos = jnp.arange(num_embeddings, dtype=jnp.float32)
    ang = pos[:, None] * inv[None, :]
    emb = jnp.concatenate([jnp.sin(ang), jnp.cos(ang)], axis=1)
    emb = emb.at[padding_idx, :].set(0.0)
    return emb


def kernel(x):
    bsz, seq_len = x.shape
    n_rows = bsz * seq_len
    table = _build_table(_PADDING_IDX + 1 + seq_len, _EMBED_DIM, _PADDING_IDX)
    table = jnp.concatenate(
        [jnp.zeros((_FRONT_PAD, _EMBED_DIM), jnp.float32), table], axis=0
    )
    xf = x.reshape(n_rows)

    info = plsc.get_sparse_core_info()
    nc, ns, lanes = info.num_cores, info.num_subcores, info.num_lanes
    nw = nc * ns
    t_per_w = seq_len // nw
    chunk = 32
    n_chunks = t_per_w // chunk
    nbuf = 2

    mesh = plsc.VectorSubcoreMesh(core_axis_name="c", subcore_axis_name="s")

    @functools.partial(
        pl.kernel,
        mesh=mesh,
        out_type=jax.ShapeDtypeStruct((n_rows + 8, _EMBED_DIM), jnp.float32),
        scratch_types=[
            pltpu.VMEM((bsz, t_per_w), jnp.int32),
            pltpu.VMEM((chunk,), jnp.int32),
            pltpu.VMEM((chunk,), jnp.int32),
            pltpu.VMEM((nbuf, chunk, _EMBED_DIM), jnp.float32),
            pltpu.VMEM((chunk, _EMBED_DIM), jnp.float32),
            pltpu.SemaphoreType.DMA,
            pltpu.SemaphoreType.DMA,
            pltpu.SemaphoreType.DMA,
        ],
    )
    def sc_kernel(table_hbm, x_hbm, out_hbm, xv, zidx, pidx, buf, zbuf, gsem, ssem, psem):
        wid = lax.axis_index("s") * nc + lax.axis_index("c")
        t0w = wid * t_per_w
        perms = [lax.iota(jnp.int32, lanes) ^ s for s in (1, 2, 4, 8)]

        for b in range(bsz):
            pltpu.sync_copy(x_hbm.at[pl.ds(b * seq_len + t0w, t_per_w)], xv.at[b])

        for i in range(chunk // lanes):
            zidx[pl.ds(i * lanes, lanes)] = jnp.full((lanes,), _ZERO_ROW, jnp.int32)
        pltpu.async_copy(table_hbm.at[zidx], zbuf, psem).wait()

        def tbl_src(ci):
            off = t0w + ci * chunk + _PADDING_IDX + 1 + _FRONT_PAD
            return table_hbm.at[pl.ds(off, chunk)]

        def out_dst(b, ci):
            return out_hbm.at[pl.ds(b * seq_len + t0w + ci * chunk, chunk)]

        def start_gather(ci):
            pltpu.make_async_copy(tbl_src(ci), buf.at[ci % nbuf], gsem).start()

        start_gather(0)
        for ci in range(n_chunks):
            pltpu.make_async_copy(tbl_src(ci), buf.at[ci % nbuf], gsem).wait()
            for b in range(bsz):
                pltpu.make_async_copy(buf.at[ci % nbuf], out_dst(b, ci), ssem).start()
            if ci + 1 < n_chunks:
                if ci >= 1:
                    for b in range(bsz):
                        pltpu.make_async_copy(
                            buf.at[(ci - 1) % nbuf], out_dst(b, ci - 1), ssem
                        ).wait()
                start_gather(ci + 1)
        for ci in (n_chunks - 2, n_chunks - 1):
            for b in range(bsz):
                pltpu.make_async_copy(buf.at[ci % nbuf], out_dst(b, ci), ssem).wait()

        for ci in range(n_chunks):
            tb = ci * chunk
            for b in range(bsz):
                acc = None
                for i in range(chunk // lanes):
                    toks = xv[b, pl.ds(tb + i * lanes, lanes)]
                    m = jnp.where(toks == _PADDING_IDX, 1, 0)
                    acc = m if acc is None else acc | m
                for perm in perms:
                    acc = jnp.maximum(
                        acc, acc.at[perm].get(mode="promise_in_bounds")
                    )
                has_pad = acc[0] > 0

                def patch(tb=tb, b=b):
                    for i in range(chunk // lanes):
                        toks = xv[b, pl.ds(tb + i * lanes, lanes)]
                        rows = lax.iota(jnp.int32, lanes) + (
                            b * seq_len + t0w + tb + i * lanes
                        )
                        pidx[pl.ds(i * lanes, lanes)] = jnp.where(
                            toks == _PADDING_IDX, rows, n_rows
                        )
                    pltpu.async_copy(zbuf, out_hbm.at[pidx], psem).wait()

                lax.cond(has_pad, patch, lambda: None)

    out = sc_kernel(table, xf)
    return out[:n_rows].reshape(bsz, seq_len, _EMBED_DIM)


# speedup vs baseline: 17.5467x; 1.3506x over previous
import functools
import math

import jax
import jax.numpy as jnp
from jax import lax
from jax.experimental import pallas as pl
from jax.experimental.pallas import tpu as pltpu
from jax.experimental.pallas import tpu_sc as plsc

_EMBED_DIM = 1024
_PADDING_IDX = 1
_FRONT_PAD = 8 - (_PADDING_IDX + 1)
_ZERO_ROW = _PADDING_IDX + _FRONT_PAD


def _build_table(num_embeddings: int, embed_dim: int, padding_idx: int):
    half = embed_dim // 2
    scale = math.log(10000.0) / (half - 1)
    inv = jnp.exp(jnp.arange(half, dtype=jnp.float32) * -scale)
    pos = jnp.arange(num_embeddings, dtype=jnp.float32)
    ang = pos[:, None] * inv[None, :]
    emb = jnp.concatenate([jnp.sin(ang), jnp.cos(ang)], axis=1)
    emb = emb.at[padding_idx, :].set(0.0)
    return emb


def kernel(x):
    bsz, seq_len = x.shape
    n_rows = bsz * seq_len
    table = _build_table(_PADDING_IDX + 1 + seq_len, _EMBED_DIM, _PADDING_IDX)
    table = jnp.concatenate(
        [jnp.zeros((_FRONT_PAD, _EMBED_DIM), jnp.float32), table], axis=0
    )
    xf = x.reshape(n_rows)

    info = plsc.get_sparse_core_info()
    nc, ns, lanes = info.num_cores, info.num_subcores, info.num_lanes
    nw = nc * ns
    t_per_w = seq_len // nw
    chunk = 32
    n_chunks = t_per_w // chunk
    nbuf = 2

    mesh = plsc.VectorSubcoreMesh(core_axis_name="c", subcore_axis_name="s")

    @functools.partial(
        pl.kernel,
        mesh=mesh,
        out_type=jax.ShapeDtypeStruct((n_rows, _EMBED_DIM), jnp.float32),
        scratch_types=[
            pltpu.VMEM((bsz, t_per_w), jnp.int32),
            pltpu.VMEM((chunk,), jnp.int32),
            pltpu.VMEM((chunk,), jnp.int32),
            pltpu.VMEM((nbuf, chunk, _EMBED_DIM), jnp.float32),
            pltpu.VMEM((chunk, _EMBED_DIM), jnp.float32),
            pltpu.SemaphoreType.DMA,
            pltpu.SemaphoreType.DMA,
            pltpu.SemaphoreType.DMA,
        ],
    )
    def sc_kernel(table_hbm, x_hbm, out_hbm, xv, zidx, pidx, buf, zbuf, gsem, ssem, psem):
        wid = lax.axis_index("s") * nc + lax.axis_index("c")
        t0w = wid * t_per_w
        perms = [lax.iota(jnp.int32, lanes) ^ s for s in (1, 2, 4, 8)]

        for b in range(bsz):
            pltpu.sync_copy(x_hbm.at[pl.ds(b * seq_len + t0w, t_per_w)], xv.at[b])

        for i in range(chunk // lanes):
            zidx[pl.ds(i * lanes, lanes)] = jnp.full((lanes,), _ZERO_ROW, jnp.int32)
        pltpu.async_copy(table_hbm.at[zidx], zbuf, psem).wait()

        def tbl_src(ci):
            off = t0w + ci * chunk + _PADDING_IDX + 1 + _FRONT_PAD
            return table_hbm.at[pl.ds(off, chunk)]

        def out_dst(b, ci):
            return out_hbm.at[pl.ds(b * seq_len + t0w + ci * chunk, chunk)]

        def start_gather(ci):
            pltpu.make_async_copy(tbl_src(ci), buf.at[ci % nbuf], gsem).start()

        start_gather(0)
        for ci in range(n_chunks):
            pltpu.make_async_copy(tbl_src(ci), buf.at[ci % nbuf], gsem).wait()
            for b in range(bsz):
                pltpu.make_async_copy(buf.at[ci % nbuf], out_dst(b, ci), ssem).start()
            if ci + 1 < n_chunks:
                if ci >= 1:
                    for b in range(bsz):
                        pltpu.make_async_copy(
                            buf.at[(ci - 1) % nbuf], out_dst(b, ci - 1), ssem
                        ).wait()
                start_gather(ci + 1)
        for ci in (n_chunks - 2, n_chunks - 1):
            for b in range(bsz):
                pltpu.make_async_copy(buf.at[ci % nbuf], out_dst(b, ci), ssem).wait()

        for ci in range(n_chunks):
            tb = ci * chunk
            for b in range(bsz):
                acc = None
                for i in range(chunk // lanes):
                    toks = xv[b, pl.ds(tb + i * lanes, lanes)]
                    m = jnp.where(toks == _PADDING_IDX, 1, 0)
                    acc = m if acc is None else acc | m
                for perm in perms:
                    acc = jnp.maximum(
                        acc, acc.at[perm].get(mode="promise_in_bounds")
                    )
                has_pad = acc[0] > 0

                def patch(tb=tb, b=b):
                    big = jnp.int32(1 << 30)
                    first = None
                    for i in range(chunk // lanes):
                        toks = xv[b, pl.ds(tb + i * lanes, lanes)]
                        rows = lax.iota(jnp.int32, lanes) + (
                            b * seq_len + t0w + tb + i * lanes
                        )
                        cand = jnp.where(toks == _PADDING_IDX, rows, big)
                        first = cand if first is None else jnp.minimum(first, cand)
                    for perm in perms:
                        first = jnp.minimum(
                            first, first.at[perm].get(mode="promise_in_bounds")
                        )
                    for i in range(chunk // lanes):
                        toks = xv[b, pl.ds(tb + i * lanes, lanes)]
                        rows = lax.iota(jnp.int32, lanes) + (
                            b * seq_len + t0w + tb + i * lanes
                        )
                        pidx[pl.ds(i * lanes, lanes)] = jnp.where(
                            toks == _PADDING_IDX, rows, first
                        )
                    pltpu.async_copy(zbuf, out_hbm.at[pidx], psem).wait()

                lax.cond(has_pad, patch, lambda: None)

    out = sc_kernel(table, xf)
    return out.reshape(bsz, seq_len, _EMBED_DIM)
